# initial kernel scaffold (unmeasured)
import jax
import jax.numpy as jnp
from jax import lax
from jax.experimental import pallas as pl
from jax.experimental.pallas import tpu as pltpu


def kernel(
    x,
):
    def body(*refs):
        pass

    out_shape = jax.ShapeDtypeStruct(..., jnp.float32)
    return pl.pallas_call(body, out_shape=out_shape)(...)


# baseline (device time: 15200 ns/iter reference)
import jax
import jax.numpy as jnp
from jax import lax
from jax.experimental import pallas as pl
from jax.experimental.pallas import tpu as pltpu

N_DEV = 4


def _bitonic_sort(x):
    n = x.shape[0]
    i = lax.broadcasted_iota(jnp.int32, x.shape, 0)
    k = 2
    while k <= n:
        j = k // 2
        while j >= 1:
            bit_j = (i & j) != 0
            bit_k = (i & k) != 0
            partner = jnp.where(
                bit_j, jnp.roll(x, j, axis=0), jnp.roll(x, -j, axis=0)
            )
            mn = jnp.minimum(x, partner)
            mx = jnp.maximum(x, partner)
            take_max = bit_j != bit_k
            x = jnp.where(take_max, mx, mn)
            j //= 2
        k *= 2
    return x


def kernel(x):
    m_per, n = x.shape

    def body(x_ref, out_ref, gather_ref, comm_ref, send_sems, recv_sems):
        my_pos = lax.axis_index("i")
        left = (my_pos - 1) % N_DEV
        right = (my_pos + 1) % N_DEV

        barrier_sem = pltpu.get_barrier_semaphore()
        for nbr in [left, right]:
            pl.semaphore_signal(
                barrier_sem, inc=1,
                device_id=(nbr,), device_id_type=pl.DeviceIdType.MESH,
            )
        pl.semaphore_wait(barrier_sem, 2)

        gather_ref[pl.ds(my_pos * m_per, m_per), :] = x_ref[:, :]
        comm_ref[0, :, :] = x_ref[:, :]

        for h in range(N_DEV - 1):
            send_slot = h % 2
            recv_slot = (h + 1) % 2
            rdma = pltpu.make_async_remote_copy(
                src_ref=comm_ref.at[send_slot],
                dst_ref=comm_ref.at[recv_slot],
                send_sem=send_sems.at[send_slot],
                recv_sem=recv_sems.at[recv_slot],
                device_id=(right,),
                device_id_type=pl.DeviceIdType.MESH,
            )
            rdma.start()
            rdma.wait()
            origin = (my_pos - h - 1) % N_DEV
            gather_ref[pl.ds(origin * m_per, m_per), :] = comm_ref[recv_slot, :, :]

        gather_ref[:, :] = _bitonic_sort(gather_ref[:, :])
        out_ref[:, :] = gather_ref[pl.ds(my_pos * m_per, m_per), :]

    return pl.pallas_call(
        body,
        out_shape=jax.ShapeDtypeStruct((m_per, n), x.dtype),
        in_specs=[pl.BlockSpec(memory_space=pltpu.VMEM)],
        out_specs=pl.BlockSpec(memory_space=pltpu.VMEM),
        scratch_shapes=[
            pltpu.VMEM((N_DEV * m_per, n), x.dtype),
            pltpu.VMEM((2, m_per, n), x.dtype),
            pltpu.SemaphoreType.DMA((2,)),
            pltpu.SemaphoreType.DMA((2,)),
        ],
        compiler_params=pltpu.CompilerParams(collective_id=0),
    )(x)


# device time: 9107 ns/iter; 1.6690x vs baseline; 1.6690x over previous
import jax
import jax.numpy as jnp
from jax import lax
from jax.experimental import pallas as pl
from jax.experimental.pallas import tpu as pltpu

N_DEV = 4


def _ce_step(x, j, k, desc=None):
    i = lax.broadcasted_iota(jnp.int32, x.shape, 0)
    bit_j = (i & j) != 0
    bit_k = (i & k) != 0
    partner = jnp.where(bit_j, jnp.roll(x, j, axis=0), jnp.roll(x, -j, axis=0))
    take_max = bit_j != bit_k
    if desc is not None:
        take_max = take_max != desc
    return jnp.where(take_max, jnp.maximum(x, partner), jnp.minimum(x, partner))


def _bitonic_sort(x, desc):
    n = x.shape[0]
    k = 2
    while k <= n:
        j = k // 2
        while j >= 1:
            x = _ce_step(x, j, k, desc)
            j //= 2
        k *= 2
    return x


def kernel(x):
    m_per, n = x.shape

    def body(x_ref, out_ref, gather_ref, send_sems, recv_sems):
        my_pos = lax.axis_index("i")

        barrier_sem = pltpu.get_barrier_semaphore()
        for d in range(1, N_DEV):
            pl.semaphore_signal(
                barrier_sem, inc=1,
                device_id=((my_pos + d) % N_DEV,),
                device_id_type=pl.DeviceIdType.MESH,
            )
        pl.semaphore_wait(barrier_sem, N_DEV - 1)

        desc = (my_pos & 1) != 0
        local = _bitonic_sort(x_ref[:, :], desc)
        my_slot = pl.ds(my_pos * m_per, m_per)
        gather_ref[my_slot, :] = local

        sends = []
        for d in range(1, N_DEV):
            rdma = pltpu.make_async_remote_copy(
                src_ref=gather_ref.at[my_slot],
                dst_ref=gather_ref.at[my_slot],
                send_sem=send_sems.at[d - 1],
                recv_sem=recv_sems.at[d - 1],
                device_id=((my_pos + d) % N_DEV,),
                device_id_type=pl.DeviceIdType.MESH,
            )
            rdma.start()
            sends.append(rdma)

        for d in range(1, N_DEV):
            origin_slot = pl.ds(((my_pos - d) % N_DEV) * m_per, m_per)
            recv = pltpu.make_async_remote_copy(
                src_ref=gather_ref.at[origin_slot],
                dst_ref=gather_ref.at[origin_slot],
                send_sem=send_sems.at[d - 1],
                recv_sem=recv_sems.at[d - 1],
                device_id=(my_pos,),
                device_id_type=pl.DeviceIdType.MESH,
            )
            recv.wait_recv()

        full = gather_ref[:, :]
        for j in (128, 64, 32, 16, 8, 4, 2, 1):
            full = _ce_step(full, j, 256)

        half = m_per * 2
        in_hi_half = (my_pos & 2) != 0
        y = jnp.where(
            in_hi_half,
            jnp.maximum(full[:half], full[half:]),
            jnp.minimum(full[:half], full[half:]),
        )
        in_hi_quarter = (my_pos & 1) != 0
        z = jnp.where(
            in_hi_quarter,
            jnp.maximum(y[:m_per], y[m_per:]),
            jnp.minimum(y[:m_per], y[m_per:]),
        )
        for j in (64, 32, 16, 8, 4, 2, 1):
            z = _ce_step(z, j, 2 * m_per)
        out_ref[:, :] = z

        for rdma in sends:
            rdma.wait_send()

    return pl.pallas_call(
        body,
        out_shape=jax.ShapeDtypeStruct((m_per, n), x.dtype),
        in_specs=[pl.BlockSpec(memory_space=pltpu.VMEM)],
        out_specs=pl.BlockSpec(memory_space=pltpu.VMEM),
        scratch_shapes=[
            pltpu.VMEM((N_DEV * m_per, n), x.dtype),
            pltpu.SemaphoreType.DMA((N_DEV - 1,)),
            pltpu.SemaphoreType.DMA((N_DEV - 1,)),
        ],
        compiler_params=pltpu.CompilerParams(collective_id=0),
    )(x)


# device time: 8040 ns/iter; 1.8905x vs baseline; 1.1327x over previous
import jax
import jax.numpy as jnp
from jax import lax
from jax.experimental import pallas as pl
from jax.experimental.pallas import tpu as pltpu

N_DEV = 4


def _gbit(b, rowi, lanehi):
    if b == 6:
        return lanehi
    jr = 1 << (b if b < 6 else b - 1)
    return (rowi & jr) != 0


def _pstep(x, rowi, lanehi, jb, kb, desc=None):
    bit_j = _gbit(jb, rowi, lanehi)
    if jb == 6:
        partner = jnp.roll(x, 64, axis=1)
    else:
        jr = 1 << (jb if jb < 6 else jb - 1)
        partner = jnp.where(
            bit_j, jnp.roll(x, jr, axis=0), jnp.roll(x, -jr, axis=0)
        )
    take_max = bit_j != _gbit(kb, rowi, lanehi)
    if desc is not None:
        take_max = take_max != desc
    return jnp.where(take_max, jnp.maximum(x, partner), jnp.minimum(x, partner))


def _iotas(shape):
    rowi = lax.broadcasted_iota(jnp.int32, shape, 0)
    lanehi = lax.broadcasted_iota(jnp.int32, shape, 1) >= 64
    return rowi, lanehi


def kernel(x):
    m_per, n = x.shape
    mp = m_per // 2

    def body(x_ref, out_ref, gather_ref, send_sems, recv_sems):
        my_pos = lax.axis_index("i")

        barrier_sem = pltpu.get_barrier_semaphore()
        for d in range(1, N_DEV):
            pl.semaphore_signal(
                barrier_sem, inc=1,
                device_id=((my_pos + d) % N_DEV,),
                device_id_type=pl.DeviceIdType.MESH,
            )
        pl.semaphore_wait(barrier_sem, N_DEV - 1)

        desc = (my_pos & 1) != 0
        local = jnp.concatenate([x_ref[:mp, :], x_ref[mp:, :]], axis=1)
        rowi_s, lanehi_s = _iotas((mp, 2 * n))
        for a in range(1, 8):
            for jb in reversed(range(a)):
                local = _pstep(local, rowi_s, lanehi_s, jb, a, desc)

        my_slot = pl.ds(my_pos * mp, mp)
        gather_ref[my_slot, :] = local

        sends = []
        for d in range(1, N_DEV):
            rdma = pltpu.make_async_remote_copy(
                src_ref=gather_ref.at[my_slot],
                dst_ref=gather_ref.at[my_slot],
                send_sem=send_sems.at[d - 1],
                recv_sem=recv_sems.at[d - 1],
                device_id=((my_pos + d) % N_DEV,),
                device_id_type=pl.DeviceIdType.MESH,
            )
            rdma.start()
            sends.append(rdma)

        for d in range(1, N_DEV):
            origin_slot = pl.ds(((my_pos - d) % N_DEV) * mp, mp)
            recv = pltpu.make_async_remote_copy(
                src_ref=gather_ref.at[origin_slot],
                dst_ref=gather_ref.at[origin_slot],
                send_sem=send_sems.at[d - 1],
                recv_sem=recv_sems.at[d - 1],
                device_id=(my_pos,),
                device_id_type=pl.DeviceIdType.MESH,
            )
            recv.wait_recv()

        full = gather_ref[:, :]
        rowi_f, lanehi_f = _iotas((N_DEV * mp, 2 * n))
        for jb in reversed(range(8)):
            full = _pstep(full, rowi_f, lanehi_f, jb, 8)

        full = _pstep(full, rowi_f, lanehi_f, 8, 9)
        in_hi_half = (my_pos & 2) != 0
        h = jnp.where(in_hi_half, full[2 * mp:], full[:2 * mp])
        in_hi_block = (my_pos & 1) != 0
        v = jnp.where(
            in_hi_block,
            jnp.maximum(h[:mp], h[mp:]),
            jnp.minimum(h[:mp], h[mp:]),
        )
        for jb in reversed(range(7)):
            v = _pstep(v, rowi_s, lanehi_s, jb, 9)

        out_ref[:mp, :] = v[:, :n]
        out_ref[mp:, :] = v[:, n:]

        for rdma in sends:
            rdma.wait_send()

    return pl.pallas_call(
        body,
        out_shape=jax.ShapeDtypeStruct((m_per, n), x.dtype),
        in_specs=[pl.BlockSpec(memory_space=pltpu.VMEM)],
        out_specs=pl.BlockSpec(memory_space=pltpu.VMEM),
        scratch_shapes=[
            pltpu.VMEM((N_DEV * mp, 2 * n), x.dtype),
            pltpu.SemaphoreType.DMA((N_DEV - 1,)),
            pltpu.SemaphoreType.DMA((N_DEV - 1,)),
        ],
        compiler_params=pltpu.CompilerParams(collective_id=0),
    )(x)


# device time: 7770 ns/iter; 1.9562x vs baseline; 1.0347x over previous
import os

import jax
import jax.numpy as jnp
from jax import lax
from jax.experimental import pallas as pl
from jax.experimental.pallas import tpu as pltpu

N_DEV = 4
_ABLATE = os.environ.get("ABLATE", "")


def _gbit(b, rowi, lanehi):
    if b == 6:
        return lanehi
    jr = 1 << (b if b < 6 else b - 1)
    return (rowi & jr) != 0


def _pstep(x, rowi, lanehi, jb, kb, desc=None):
    bit_j = _gbit(jb, rowi, lanehi)
    if jb == 6:
        partner = jnp.roll(x, 64, axis=1)
    else:
        jr = 1 << (jb if jb < 6 else jb - 1)
        partner = jnp.where(
            bit_j, jnp.roll(x, jr, axis=0), jnp.roll(x, -jr, axis=0)
        )
    take_max = bit_j != _gbit(kb, rowi, lanehi)
    if desc is not None:
        take_max = take_max != desc
    return jnp.where(take_max, jnp.maximum(x, partner), jnp.minimum(x, partner))


def _iotas(shape):
    rowi = lax.broadcasted_iota(jnp.int32, shape, 0)
    lanehi = lax.broadcasted_iota(jnp.int32, shape, 1) >= 64
    return rowi, lanehi


def kernel(x):
    m_per, n = x.shape
    mp = m_per // 2

    def body(x_ref, out_ref, gather_ref, send_sems, recv_sems):
        if _ABLATE == "copy":
            out_ref[:, :] = x_ref[:, :]
            return
        my_pos = lax.axis_index("i")

        barrier_sem = None
        if _ABLATE not in ("compute", "autobar"):
            barrier_sem = pltpu.get_barrier_semaphore()
            for d in range(1, N_DEV) if _ABLATE != "nobar" else ():
                pl.semaphore_signal(
                    barrier_sem, inc=1,
                    device_id=((my_pos + d) % N_DEV,),
                    device_id_type=pl.DeviceIdType.MESH,
                )

        desc = (my_pos & 1) != 0
        local = jnp.concatenate([x_ref[:mp, :], x_ref[mp:, :]], axis=1)
        rowi_s, lanehi_s = _iotas((mp, 2 * n))
        if _ABLATE not in ("comm", "barrier"):
            for a in range(1, 8):
                for jb in reversed(range(a)):
                    local = _pstep(local, rowi_s, lanehi_s, jb, a, desc)

        my_slot = pl.ds(my_pos * mp, mp)
        gather_ref[my_slot, :] = local

        if _ABLATE not in ("compute", "autobar"):
            pl.semaphore_wait(barrier_sem, 0 if _ABLATE == "nobar" else N_DEV - 1)

        if _ABLATE == "barrier":
            out_ref[:mp, :] = local[:, :n]
            out_ref[mp:, :] = local[:, n:]
            return

        sends = []
        for d in (2, 1, 3) if _ABLATE != "compute" else ():
            rdma = pltpu.make_async_remote_copy(
                src_ref=gather_ref.at[my_slot],
                dst_ref=gather_ref.at[my_slot],
                send_sem=send_sems.at[d - 1],
                recv_sem=recv_sems.at[d - 1],
                device_id=((my_pos + d) % N_DEV,),
                device_id_type=pl.DeviceIdType.MESH,
            )
            rdma.start()
            sends.append(rdma)

        def wait_from(d):
            origin_slot = pl.ds(((my_pos - d) % N_DEV) * mp, mp)
            pltpu.make_async_remote_copy(
                src_ref=gather_ref.at[origin_slot],
                dst_ref=gather_ref.at[origin_slot],
                send_sem=send_sems.at[d - 1],
                recv_sem=recv_sems.at[d - 1],
                device_id=(my_pos,),
                device_id_type=pl.DeviceIdType.MESH,
            ).wait_recv()

        if _ABLATE != "compute":
            wait_from(1)
            wait_from(3)

        if _ABLATE == "comm":
            wait_from(2)
            out_ref[:mp, :] = gather_ref[my_slot, :][:, :n]
            out_ref[mp:, :] = gather_ref[my_slot, :][:, n:]
            for rdma in sends:
                rdma.wait_send()
            return

        in_hi_pair = (my_pos & 2) != 0
        rowi_p, lanehi_p = _iotas((2 * mp, 2 * n))

        def merge_pair(block, flip):
            for jb in reversed(range(8)):
                block = _pstep(block, rowi_p, lanehi_p, jb, 8, flip)
            return block

        pa = jnp.where(in_hi_pair, gather_ref[2 * mp:, :], gather_ref[:2 * mp, :])
        pa = merge_pair(pa, in_hi_pair)

        if _ABLATE != "compute":
            wait_from(2)
        pb = jnp.where(in_hi_pair, gather_ref[:2 * mp, :], gather_ref[2 * mp:, :])
        pb = merge_pair(pb, jnp.logical_not(in_hi_pair))

        w = jnp.where(in_hi_pair, jnp.maximum(pa, pb), jnp.minimum(pa, pb))
        in_hi_block = (my_pos & 1) != 0
        v = jnp.where(
            in_hi_block,
            jnp.maximum(w[:mp], w[mp:]),
            jnp.minimum(w[:mp], w[mp:]),
        )
        for jb in reversed(range(7)):
            v = _pstep(v, rowi_s, lanehi_s, jb, 9)

        out_ref[:mp, :] = v[:, :n]
        out_ref[mp:, :] = v[:, n:]

        for rdma in sends:
            rdma.wait_send()

    return pl.pallas_call(
        body,
        out_shape=jax.ShapeDtypeStruct((m_per, n), x.dtype),
        in_specs=[pl.BlockSpec(memory_space=pltpu.VMEM)],
        out_specs=pl.BlockSpec(memory_space=pltpu.VMEM),
        scratch_shapes=[
            pltpu.VMEM((N_DEV * mp, 2 * n), x.dtype),
            pltpu.SemaphoreType.DMA((N_DEV - 1,)),
            pltpu.SemaphoreType.DMA((N_DEV - 1,)),
        ],
        compiler_params=(
            None
            if _ABLATE in ("copy", "compute", "autobar")
            else pltpu.CompilerParams(collective_id=0)
        ),
    )(x)
